# baseline (speedup 1.0000x reference)
import functools

import jax
import jax.numpy as jnp
from jax import lax
from jax.experimental import pallas as pl
from jax.experimental.pallas import tpu as pltpu
from jax.experimental.pallas import tpu_sc as plsc

M = 100000
B = 16384
D = 128
NW = 32
S = 3136
S_LAST = M - S * (NW - 1)
T = S
CH = 64
WB = S + CH
CPC = 128
NSEG = S // CPC
NSEG_LAST = S_LAST // CPC
TAIL_MAIN = S - NSEG * CPC
TAIL_LAST = S_LAST - NSEG_LAST * CPC
NCHUNK = B // 16
NSEG_DEDUP = 21
DPS = 48
BF = 8


def _dedup_chunks(idx_v, table_v, row_lo, n_own, iota, base, chunks):
  for group_start in range(0, len(chunks), BF):
    group = chunks[group_start:group_start + BF]
    ivecs = [idx_v[pl.ds((base + c) * 16, 16)] for c in group]
    locals_ = [ivec - row_lo for ivec in ivecs]
    valids = [(l >= 0) & (l < n_own) for l in locals_]
    lasts = [plsc.scan_count(ivec, mask=v)[1]
             for ivec, v in zip(ivecs, valids)]
    for cc, l, v, last in zip(group, locals_, valids, lasts):
      m = v & last
      l_c = jnp.clip(l, 0, T - 1)
      plsc.store_scatter(table_v, [l_c], (base + cc) * 16 + iota, mask=m)


def _body(idx_hbm, val_hbm, mem_hbm, out_hbm, idx_v, table_v, nodes_v,
          gidx_v, nchunk_n0, nchunk_n1, rows_v0, rows_v1, cbuf0, cbuf1,
          cbuf2, gsem0, gsem1, gsem2, ssem0, ssem1, ssem2, wgsem, wssem0,
          wssem1):
  c = lax.axis_index("c")
  s = lax.axis_index("s")
  wid = s * 2 + c
  row_lo = wid * S
  is_last = wid == NW - 1
  n_own = jnp.where(is_last, S_LAST, S)
  nseg = jnp.where(is_last, NSEG_LAST, NSEG)
  cbufs = (cbuf0, cbuf1, cbuf2)
  gsems = (gsem0, gsem1, gsem2)
  ssems = (ssem0, ssem1, ssem2)
  nchunk_ns = (nchunk_n0, nchunk_n1)
  rows_vs = (rows_v0, rows_v1)
  wssems = (wssem0, wssem1)
  iota = lax.iota(jnp.int32, 16)

  def gather_cp(par, off):
    return pltpu.make_async_copy(
        mem_hbm.at[pl.ds(off, CPC)], cbufs[par], gsems[par])

  def scatter_cp(par, off):
    return pltpu.make_async_copy(
        cbufs[par], out_hbm.at[pl.ds(off, CPC)], ssems[par])

  pltpu.sync_copy(idx_hbm, idx_v)

  gather_cp(0, row_lo).start()

  minus1 = jnp.full((16,), -1, jnp.int32)

  def zero_body(i, carry):
    for u in range(4):
      table_v[pl.ds((i * 4 + u) * 16, 16)] = minus1
    return carry

  lax.fori_loop(0, T // 16 // 4, zero_body, 0)

  def seg_body(i, carry):
    off = row_lo + i * CPC
    for par in range(3):
      @pl.when(lax.rem(i, 3) == par)
      def _():
        pnext = (par + 1) % 3
        gather_cp(par, off).wait()
        scatter_cp(par, off).start()

        @pl.when(i >= 2)
        def _():
          scatter_cp(pnext, off - 2 * CPC).wait()

        @pl.when(i + 1 < nseg)
        def _():
          gather_cp(pnext, off + CPC).start()

    @pl.when(i < NSEG_DEDUP)
    def _():
      _dedup_chunks(idx_v, table_v, row_lo, n_own, iota, i * DPS,
                    list(range(DPS)))
    return carry

  lax.fori_loop(0, nseg, seg_body, 0)

  scatter_cp(1, row_lo).wait()
  scatter_cp(2, row_lo).wait()

  @pl.when(jnp.logical_not(is_last))
  def _():
    off = row_lo + NSEG * CPC
    pltpu.make_async_copy(mem_hbm.at[pl.ds(off, TAIL_MAIN)],
                          cbuf0.at[pl.ds(0, TAIL_MAIN)], gsem0).start()
    pltpu.make_async_copy(mem_hbm.at[pl.ds(off, TAIL_MAIN)],
                          cbuf0.at[pl.ds(0, TAIL_MAIN)], gsem0).wait()
    pltpu.make_async_copy(cbuf0.at[pl.ds(0, TAIL_MAIN)],
                          out_hbm.at[pl.ds(off, TAIL_MAIN)], ssem0).start()
    pltpu.make_async_copy(cbuf0.at[pl.ds(0, TAIL_MAIN)],
                          out_hbm.at[pl.ds(off, TAIL_MAIN)], ssem0).wait()

  @pl.when(is_last)
  def _():
    off = row_lo + NSEG_LAST * CPC
    pltpu.make_async_copy(mem_hbm.at[pl.ds(off, TAIL_LAST)],
                          cbuf0.at[pl.ds(0, TAIL_LAST)], gsem0).start()
    pltpu.make_async_copy(mem_hbm.at[pl.ds(off, TAIL_LAST)],
                          cbuf0.at[pl.ds(0, TAIL_LAST)], gsem0).wait()
    pltpu.make_async_copy(cbuf0.at[pl.ds(0, TAIL_LAST)],
                          out_hbm.at[pl.ds(off, TAIL_LAST)], ssem0).start()
    pltpu.make_async_copy(cbuf0.at[pl.ds(0, TAIL_LAST)],
                          out_hbm.at[pl.ds(off, TAIL_LAST)], ssem0).wait()

  _dedup_chunks(idx_v, table_v, row_lo, n_own, iota, NSEG_DEDUP * DPS,
                list(range(NCHUNK - NSEG_DEDUP * DPS)))

  def extract_body(t, off):
    tv = table_v[pl.ds(t * 16, 16)]
    m = tv >= 0
    nodes = (row_lo + t * 16) + iota
    plsc.store_compressed(nodes_v.at[pl.ds(off, 16)], nodes, mask=m)
    plsc.store_compressed(gidx_v.at[pl.ds(off, 16)], tv, mask=m)
    return off + jnp.sum(m.astype(jnp.int32))

  nwin = lax.fori_loop(0, T // 16, extract_body, jnp.int32(0))

  @pl.when(nwin > 0)
  def _():
    lane0 = (iota == 0).astype(jnp.int32)
    n0 = jnp.sum(nodes_v[pl.ds(0, 16)] * lane0)
    g0 = jnp.sum(gidx_v[pl.ds(0, 16)] * lane0)
    npad = jnp.zeros((16,), jnp.int32) + n0
    gpad = jnp.zeros((16,), jnp.int32) + g0
    for k in range(CH // 16):
      nodes_v[pl.ds(nwin + k * 16, 16)] = npad
      gidx_v[pl.ds(nwin + k * 16, 16)] = gpad

  nchunks = (nwin + CH - 1) // CH

  def chunk_body(ci, carry):
    off = ci * CH
    for par in range(2):
      @pl.when(lax.rem(ci, 2) == par)
      def _():
        nb = nchunk_ns[par]
        rb = rows_vs[par]

        @pl.when(ci >= 2)
        def _():
          pltpu.make_async_copy(rb, out_hbm.at[nb], wssems[par]).wait()

        for k in range(CH // 16):
          nb[pl.ds(k * 16, 16)] = nodes_v[pl.ds(off + k * 16, 16)]
        pltpu.async_copy(val_hbm.at[gidx_v.at[pl.ds(off, CH)]], rb,
                         wgsem).wait()
        pltpu.make_async_copy(rb, out_hbm.at[nb], wssems[par]).start()
    return carry

  lax.fori_loop(0, nchunks, chunk_body, 0)

  @pl.when(nchunks >= 1)
  def _():
    par = lax.rem(nchunks - 1, 2)
    for p in range(2):
      @pl.when(par == p)
      def _():
        pltpu.make_async_copy(rows_vs[p], out_hbm.at[nchunk_ns[p]],
                              wssems[p]).wait()

  @pl.when(nchunks >= 2)
  def _():
    par = lax.rem(nchunks - 2, 2)
    for p in range(2):
      @pl.when(par == p)
      def _():
        pltpu.make_async_copy(rows_vs[p], out_hbm.at[nchunk_ns[p]],
                              wssems[p]).wait()


_agg = functools.partial(
    pl.kernel,
    out_type=jax.ShapeDtypeStruct((M, D), jnp.float32),
    mesh=plsc.VectorSubcoreMesh(core_axis_name="c", subcore_axis_name="s"),
    compiler_params=pltpu.CompilerParams(needs_layout_passes=False),
    scratch_types=[
        pltpu.VMEM((B,), jnp.int32),
        pltpu.VMEM((T,), jnp.int32),
        pltpu.VMEM((WB,), jnp.int32),
        pltpu.VMEM((WB,), jnp.int32),
        pltpu.VMEM((CH,), jnp.int32),
        pltpu.VMEM((CH,), jnp.int32),
        pltpu.VMEM((CH, D), jnp.float32),
        pltpu.VMEM((CH, D), jnp.float32),
        pltpu.VMEM((CPC, D), jnp.float32),
        pltpu.VMEM((CPC, D), jnp.float32),
        pltpu.VMEM((CPC, D), jnp.float32),
        pltpu.SemaphoreType.DMA,
        pltpu.SemaphoreType.DMA,
        pltpu.SemaphoreType.DMA,
        pltpu.SemaphoreType.DMA,
        pltpu.SemaphoreType.DMA,
        pltpu.SemaphoreType.DMA,
        pltpu.SemaphoreType.DMA,
        pltpu.SemaphoreType.DMA,
        pltpu.SemaphoreType.DMA,
    ],
)(_body)


def kernel(mem, idx, val):
  idx32 = idx.astype(jnp.int32)
  return _agg(idx32, val, mem)


# speedup vs baseline: 1.1895x; 1.1895x over previous
import functools

import jax
import jax.numpy as jnp
from jax import lax
from jax.experimental import pallas as pl
from jax.experimental.pallas import tpu as pltpu
from jax.experimental.pallas import tpu_sc as plsc

M = 100000
B = 16384
D = 128
NW = 32
S = 3136
S_LAST = M - S * (NW - 1)
T = S
CH = 64
WB = S + CH
NCHUNK = B // 16
DPS = 64
BF = 8


def _dedup_chunks(idx_v, table_v, row_lo, n_own, iota, base, chunks):
  for group_start in range(0, len(chunks), BF):
    group = chunks[group_start:group_start + BF]
    ivecs = [idx_v[pl.ds((base + c) * 16, 16)] for c in group]
    locals_ = [ivec - row_lo for ivec in ivecs]
    valids = [(l >= 0) & (l < n_own) for l in locals_]
    lasts = [plsc.scan_count(ivec, mask=v)[1]
             for ivec, v in zip(ivecs, valids)]
    for cc, l, v, last in zip(group, locals_, valids, lasts):
      m = v & last
      l_c = jnp.clip(l, 0, T - 1)
      plsc.store_scatter(table_v, [l_c], (base + cc) * 16 + iota, mask=m)


def _body(idx_hbm, val_hbm, out_hbm, idx_v, table_v, nodes_v, gidx_v,
          nchunk_n0, nchunk_n1, rows_v0, rows_v1, wgsem, wssem0, wssem1):
  c = lax.axis_index("c")
  s = lax.axis_index("s")
  wid = s * 2 + c
  row_lo = wid * S
  n_own = jnp.where(wid == NW - 1, S_LAST, S)
  nchunk_ns = (nchunk_n0, nchunk_n1)
  rows_vs = (rows_v0, rows_v1)
  wssems = (wssem0, wssem1)
  iota = lax.iota(jnp.int32, 16)

  pltpu.sync_copy(idx_hbm, idx_v)

  minus1 = jnp.full((16,), -1, jnp.int32)

  def zero_body(i, carry):
    for u in range(4):
      table_v[pl.ds((i * 4 + u) * 16, 16)] = minus1
    return carry

  lax.fori_loop(0, T // 16 // 4, zero_body, 0)

  def scan_body(i, carry):
    _dedup_chunks(idx_v, table_v, row_lo, n_own, iota, i * DPS,
                  list(range(DPS)))
    return carry

  lax.fori_loop(0, NCHUNK // DPS, scan_body, 0)

  def extract_body(t, off):
    tv = table_v[pl.ds(t * 16, 16)]
    m = tv >= 0
    nodes = (row_lo + t * 16) + iota
    plsc.store_compressed(nodes_v.at[pl.ds(off, 16)], nodes, mask=m)
    plsc.store_compressed(gidx_v.at[pl.ds(off, 16)], tv, mask=m)
    return off + jnp.sum(m.astype(jnp.int32))

  nwin = lax.fori_loop(0, T // 16, extract_body, jnp.int32(0))

  @pl.when(nwin > 0)
  def _():
    lane0 = (iota == 0).astype(jnp.int32)
    n0 = jnp.sum(nodes_v[pl.ds(0, 16)] * lane0)
    g0 = jnp.sum(gidx_v[pl.ds(0, 16)] * lane0)
    npad = jnp.zeros((16,), jnp.int32) + n0
    gpad = jnp.zeros((16,), jnp.int32) + g0
    for k in range(CH // 16):
      nodes_v[pl.ds(nwin + k * 16, 16)] = npad
      gidx_v[pl.ds(nwin + k * 16, 16)] = gpad

  nchunks = (nwin + CH - 1) // CH

  def chunk_body(ci, carry):
    off = ci * CH
    for par in range(2):
      @pl.when(lax.rem(ci, 2) == par)
      def _():
        nb = nchunk_ns[par]
        rb = rows_vs[par]

        @pl.when(ci >= 2)
        def _():
          pltpu.make_async_copy(rb, out_hbm.at[nb], wssems[par]).wait()

        for k in range(CH // 16):
          nb[pl.ds(k * 16, 16)] = nodes_v[pl.ds(off + k * 16, 16)]
        pltpu.async_copy(val_hbm.at[gidx_v.at[pl.ds(off, CH)]], rb,
                         wgsem).wait()
        pltpu.make_async_copy(rb, out_hbm.at[nb], wssems[par]).start()
    return carry

  lax.fori_loop(0, nchunks, chunk_body, 0)

  @pl.when(nchunks >= 1)
  def _():
    par = lax.rem(nchunks - 1, 2)
    for p in range(2):
      @pl.when(par == p)
      def _():
        pltpu.make_async_copy(rows_vs[p], out_hbm.at[nchunk_ns[p]],
                              wssems[p]).wait()

  @pl.when(nchunks >= 2)
  def _():
    par = lax.rem(nchunks - 2, 2)
    for p in range(2):
      @pl.when(par == p)
      def _():
        pltpu.make_async_copy(rows_vs[p], out_hbm.at[nchunk_ns[p]],
                              wssems[p]).wait()


_agg = functools.partial(
    pl.kernel,
    out_type=(),
    mesh=plsc.VectorSubcoreMesh(core_axis_name="c", subcore_axis_name="s"),
    compiler_params=pltpu.CompilerParams(needs_layout_passes=False),
    scratch_types=[
        pltpu.VMEM((B,), jnp.int32),
        pltpu.VMEM((T,), jnp.int32),
        pltpu.VMEM((WB,), jnp.int32),
        pltpu.VMEM((WB,), jnp.int32),
        pltpu.VMEM((CH,), jnp.int32),
        pltpu.VMEM((CH,), jnp.int32),
        pltpu.VMEM((CH, D), jnp.float32),
        pltpu.VMEM((CH, D), jnp.float32),
        pltpu.SemaphoreType.DMA,
        pltpu.SemaphoreType.DMA,
        pltpu.SemaphoreType.DMA,
    ],
)(_body)


def kernel(mem, idx, val):
  idx32 = idx.astype(jnp.int32)
  out_ref = jax.new_ref(mem)
  _agg(idx32, val, out_ref)
  return out_ref[...]
